# baseline (speedup 1.0000x reference)
import functools

import jax
import jax.numpy as jnp
from jax import lax
from jax.experimental import pallas as pl
from jax.experimental.pallas import tpu as pltpu
from jax.experimental.pallas import tpu_sc as plsc

_B, _N, _CIN, _COUT = 8, 131072, 20, 13
_BLK = 16384

_TGT_SRCS = [
    [], [1], [0], [], [], [8], [7], [6, 12], [4], [5], [9], [],
    [2, 3, 10, 11, 13, 14, 15, 16, 17, 18, 19],
]
_K = 4

_NC, _NS = 2, 16
_NW = _NC * _NS
_COLS_W = _N // _NW
_WCH = 4096


def _sc_body(in_hbm, out_hbm, in_v, ninf_v):
    wid = lax.axis_index("s") * _NC + lax.axis_index("c")
    off = wid * _COLS_W

    neg = jnp.full((16,), -jnp.inf, dtype=jnp.float32)
    for j in range(_B * _WCH // 16):
        r, k = divmod(j * 16, _WCH)
        ninf_v[r, pl.ds(k, 16)] = neg

    pltpu.sync_copy(in_hbm.at[pl.ds(0, 2), :, pl.ds(off, _WCH)], in_v)
    pltpu.sync_copy(ninf_v, out_hbm.at[0, :, pl.ds(off, _WCH)])
    pltpu.sync_copy(in_v.at[1], out_hbm.at[1, :, pl.ds(off, _WCH)])
    pltpu.sync_copy(in_v.at[0], out_hbm.at[2, :, pl.ds(off, _WCH)])
    pltpu.sync_copy(ninf_v, out_hbm.at[3, :, pl.ds(off, _WCH)])


def _tc_body(x_ref, o_ref):
    for i, srcs in enumerate(_TGT_SRCS[_K:]):
        if not srcs:
            o_ref[i] = jnp.full((_B, _BLK), -jnp.inf, dtype=jnp.float32)
        else:
            acc = [x_ref[s] for s in srcs]
            while len(acc) > 1:
                acc = [jnp.maximum(a, b) for a, b in zip(acc[::2], acc[1::2])] + (
                    [acc[-1]] if len(acc) % 2 else [])
            o_ref[i] = acc[0]


@functools.partial(jax.jit, static_argnums=())
def kernel(logits):
    xt = jnp.transpose(logits, (2, 0, 1))

    sc_run = pl.kernel(
        _sc_body,
        out_type=jax.ShapeDtypeStruct((_K, _B, _N), jnp.float32),
        mesh=plsc.VectorSubcoreMesh(core_axis_name="c", subcore_axis_name="s"),
        compiler_params=pltpu.CompilerParams(
            needs_layout_passes=False, use_tc_tiling_on_sc=False),
        scratch_types=[
            pltpu.VMEM((2, _B, _WCH), jnp.float32),
            pltpu.VMEM((_B, _WCH), jnp.float32),
        ],
    )
    sc_out = sc_run(xt)

    tc_out = pl.pallas_call(
        _tc_body,
        grid=(_N // _BLK,),
        in_specs=[pl.BlockSpec((_CIN, _B, _BLK), lambda i: (0, 0, i))],
        out_specs=pl.BlockSpec((_COUT - _K, _B, _BLK), lambda i: (0, 0, i)),
        out_shape=jax.ShapeDtypeStruct((_COUT - _K, _B, _N), jnp.float32),
    )(xt)

    out = jnp.concatenate([sc_out, tc_out], axis=0)
    return jnp.transpose(out, (1, 2, 0))


# speedup vs baseline: 3.9399x; 3.9399x over previous
import functools

import jax
import jax.numpy as jnp
from jax.experimental import pallas as pl

_B, _N, _CIN, _COUT = 8, 131072, 20, 13
_BLK = 16384

_TGT_SRCS = [
    [], [1], [0], [], [], [8], [7], [6, 12], [4], [5], [9], [],
    [2, 3, 10, 11, 13, 14, 15, 16, 17, 18, 19],
]


def _tc_body(x_ref, o_ref):
    for t, srcs in enumerate(_TGT_SRCS):
        if not srcs:
            o_ref[t] = jnp.full((_B, _BLK), -jnp.inf, dtype=jnp.float32)
        else:
            acc = [x_ref[s] for s in srcs]
            while len(acc) > 1:
                acc = [jnp.maximum(a, b) for a, b in zip(acc[::2], acc[1::2])] + (
                    [acc[-1]] if len(acc) % 2 else [])
            o_ref[t] = acc[0]


@functools.partial(jax.jit, static_argnums=())
def kernel(logits):
    xt = jnp.transpose(logits, (2, 0, 1))
    out = pl.pallas_call(
        _tc_body,
        grid=(_N // _BLK,),
        in_specs=[pl.BlockSpec((_CIN, _B, _BLK), lambda i: (0, 0, i))],
        out_specs=pl.BlockSpec((_COUT, _B, _BLK), lambda i: (0, 0, i)),
        out_shape=jax.ShapeDtypeStruct((_COUT, _B, _N), jnp.float32),
    )(xt)
    return jnp.transpose(out, (1, 2, 0))
